# Initial kernel scaffold; baseline (speedup 1.0000x reference)
import jax
import jax.numpy as jnp
from jax.experimental import pallas as pl


def kernel(x, emb1_tables, pair_emb1, pair_emb2, W, b):
    raise NotImplementedError("write your pallas kernel here")


# speedup vs baseline: 1.3221x; 1.3221x over previous
import functools

import jax
import jax.numpy as jnp
import numpy as np
from jax import lax
from jax.experimental import pallas as pl
from jax.experimental.pallas import tpu as pltpu
from jax.experimental.pallas import tpu_sc as plsc

F_FIELDS = 26
EMB = 64
VOCAB = 1000
_pairs = [(i, j) for i in range(F_FIELDS) for j in range(i, F_FIELDS)]
NPAIR = len(_pairs)
NSEG = F_FIELDS + NPAIR
B = 4096
NW = 32
MAX_SLOTS = (NSEG + NW - 1) // NW
NTP = NW * MAX_SLOTS
CH = 128
NCHUNK = B // CH
CGRP = CH // 16


def _build_task_meta():
    meta = np.full((NW, MAX_SLOTS, 16), -1, dtype=np.int32)
    tasks = []
    for f in range(F_FIELDS):
        tasks.append((f, -1, f, f))
    for p, (i, j) in enumerate(_pairs):
        tasks.append((i, j, p, F_FIELDS + p))
    for t, (c1, c2, tbl, seg) in enumerate(tasks):
        meta[t % NW, t // NW, :4] = (c1, c2, tbl, seg)
    return meta


_TASK_META = _build_task_meta()


def _sc_partials(xT3, emb1_tables, pair_emb1, pair_emb2f, Wseg, meta):
    mesh = plsc.VectorSubcoreMesh(core_axis_name="c", subcore_axis_name="s")

    @functools.partial(
        pl.kernel,
        out_type=jax.ShapeDtypeStruct((NTP, 2, B), jnp.float32),
        mesh=mesh,
        compiler_params=pltpu.CompilerParams(
            use_tc_tiling_on_sc=False, needs_layout_passes=False),
        scratch_types=[
            pltpu.VMEM((VOCAB, EMB), jnp.float32),
            pltpu.VMEM((CH, EMB), jnp.float32),
            pltpu.VMEM((CH,), jnp.int32),
            pltpu.VMEM((CH,), jnp.int32),
            pltpu.VMEM((CH,), jnp.int32),
            pltpu.VMEM((1, 2 * EMB), jnp.float32),
            pltpu.VMEM((2, CH), jnp.float32),
            pltpu.VMEM((MAX_SLOTS, 16), jnp.int32),
            pltpu.SemaphoreType.DMA,
        ],
    )
    def k(xT_h, e1_h, p1_h, p2f_h, w_h, meta_h, out_h,
          t1, g2, idx1, idx2, idx2f, wv, acc, ms, sem):
        wid = lax.axis_index("s") * 2 + lax.axis_index("c")
        pltpu.sync_copy(meta_h.at[wid], ms)
        zeros16 = jnp.zeros((16,), jnp.float32)
        one16 = jnp.ones((16,), jnp.int32)
        iota16 = lax.iota(jnp.int32, 16)

        def run_chunks(tid, col1, col2, tbl, pairwise):
            wch = [wv[0, pl.ds(c * 16, 16)] for c in range(8)]

            def chunk_body(ci, _):
                pltpu.sync_copy(xT_h.at[col1, 0, pl.ds(ci * CH, CH)], idx1)
                if pairwise:
                    pltpu.sync_copy(xT_h.at[col2, 0, pl.ds(ci * CH, CH)], idx2)
                    base = tbl * VOCAB

                    def fl_body(g, _f):
                        idx2f[pl.ds(g * 16, 16)] = idx2[pl.ds(g * 16, 16)] + base
                        return _f

                    lax.fori_loop(0, CGRP, fl_body, None)
                    pltpu.async_copy(p2f_h.at[idx2f], g2, sem).wait()

                def group_body(g, _g):
                    rows1 = idx1[pl.ds(g * 16, 16)]
                    av0 = jnp.zeros((16,), jnp.float32)
                    av1 = jnp.zeros((16,), jnp.float32)
                    ce = jnp.zeros((16,), jnp.int32)
                    if pairwise:
                        rows2 = iota16 + g * 16
                    for e in range(EMB):
                        v1 = plsc.load_gather(t1, [rows1, ce])
                        if pairwise:
                            v2 = plsc.load_gather(g2, [rows2, ce])
                            prod = jnp.maximum(v1 * v2, 0.0)
                        else:
                            prod = jnp.maximum(v1, 0.0)
                        c, l = divmod(e, 16)
                        av0 = av0 + prod * wch[c][l]
                        av1 = av1 + prod * wch[c + 4][l]
                        ce = ce + one16
                    acc[0, pl.ds(g * 16, 16)] = av0
                    acc[1, pl.ds(g * 16, 16)] = av1
                    return _g

                lax.fori_loop(0, CGRP, group_body, None)
                pltpu.sync_copy(acc, out_h.at[tid, :, pl.ds(ci * CH, CH)])
                return _

            lax.fori_loop(0, NCHUNK, chunk_body, None)

        def task_body(slot, _):
            mrow = ms[slot, pl.ds(0, 16)]
            col1 = mrow[0]
            col2 = mrow[1]
            tbl = mrow[2]
            seg = mrow[3]
            tid = slot * NW + wid

            @pl.when(seg >= 0)
            def _run():
                pltpu.sync_copy(w_h.at[seg], wv)

                @pl.when(col2 >= 0)
                def _pair():
                    pltpu.sync_copy(p1_h.at[tbl], t1)
                    run_chunks(tid, col1, col2, tbl, True)

                @pl.when(col2 < 0)
                def _field():
                    pltpu.sync_copy(e1_h.at[tbl], t1)
                    run_chunks(tid, col1, col2, tbl, False)

            @pl.when(seg < 0)
            def _idle():
                def zg(g, _g):
                    acc[0, pl.ds(g * 16, 16)] = zeros16
                    acc[1, pl.ds(g * 16, 16)] = zeros16
                    return _g

                lax.fori_loop(0, CGRP, zg, None)

                def zc(ci, _c):
                    pltpu.sync_copy(acc, out_h.at[tid, :, pl.ds(ci * CH, CH)])
                    return _c

                lax.fori_loop(0, NCHUNK, zc, None)

            return _

        lax.fori_loop(0, MAX_SLOTS, task_body, None)

    return k(xT3, emb1_tables, pair_emb1, pair_emb2f, Wseg, meta)


def _tc_finish(partials, bias):
    nblk = 32
    nsteps = NTP // nblk

    def body(p_ref, b_ref, o_ref):
        i = pl.program_id(0)

        @pl.when(i == 0)
        def _init():
            o_ref[...] = jnp.zeros_like(o_ref)

        o_ref[...] += jnp.sum(p_ref[...], axis=0)

        @pl.when(i == nsteps - 1)
        def _fin():
            o_ref[...] = jax.nn.sigmoid(o_ref[...] + b_ref[...])

    return pl.pallas_call(
        body,
        grid=(nsteps,),
        in_specs=[
            pl.BlockSpec((nblk, 2, B), lambda i: (i, 0, 0)),
            pl.BlockSpec((2, 1), lambda i: (0, 0)),
        ],
        out_specs=pl.BlockSpec((2, B), lambda i: (0, 0)),
        out_shape=jax.ShapeDtypeStruct((2, B), jnp.float32),
    )(partials, bias)


def kernel(x, emb1_tables, pair_emb1, pair_emb2, W, b):
    xT3 = x.astype(jnp.int32).T.reshape(F_FIELDS, 1, B)
    Wseg = W.reshape(2, NSEG, EMB).transpose(1, 0, 2).reshape(NSEG, 1, 2 * EMB)
    pair_emb2f = pair_emb2.reshape(NPAIR * VOCAB, EMB)
    meta = jnp.asarray(_TASK_META)
    partials = _sc_partials(xT3, emb1_tables, pair_emb1, pair_emb2f, Wseg, meta)
    y = _tc_finish(partials, b.reshape(2, 1))
    return (y[0], y[1])


# speedup vs baseline: 1.5494x; 1.1719x over previous
import functools

import jax
import jax.numpy as jnp
import numpy as np
from jax import lax
from jax.experimental import pallas as pl
from jax.experimental.pallas import tpu as pltpu
from jax.experimental.pallas import tpu_sc as plsc

F_FIELDS = 26
EMB = 64
VOCAB = 1000
_pairs = [(i, j) for i in range(F_FIELDS) for j in range(i, F_FIELDS)]
NPAIR = len(_pairs)
NSEG = F_FIELDS + NPAIR
B = 4096
NW = 32
MAX_SLOTS = (NSEG + NW - 1) // NW
NTP = NW * MAX_SLOTS
CH = 128
NCHUNK = B // CH
CGRP = CH // 16
DEPTH = 4


def _build_task_meta():
    meta = np.full((NW, MAX_SLOTS, 16), -1, dtype=np.int32)
    tasks = []
    for f in range(F_FIELDS):
        tasks.append((f, -1, f, f))
    for p, (i, j) in enumerate(_pairs):
        tasks.append((i, j, p, F_FIELDS + p))
    for t, (c1, c2, tbl, seg) in enumerate(tasks):
        meta[t % NW, t // NW, :4] = (c1, c2, tbl, seg)
    return meta


_TASK_META = _build_task_meta()


def _sc_partials(xT3, emb1_tables, pair_emb1, pair_emb2f, Wseg, meta):
    mesh = plsc.VectorSubcoreMesh(core_axis_name="c", subcore_axis_name="s")

    @functools.partial(
        pl.kernel,
        out_type=jax.ShapeDtypeStruct((NTP, 2, B), jnp.float32),
        mesh=mesh,
        compiler_params=pltpu.CompilerParams(
            use_tc_tiling_on_sc=False, needs_layout_passes=False),
        scratch_types=[
            pltpu.VMEM((VOCAB, EMB), jnp.float32),
            [pltpu.VMEM((CH, EMB), jnp.float32) for _ in range(DEPTH)],
            pltpu.VMEM((NCHUNK, CH), jnp.int32),
            pltpu.VMEM((NCHUNK, CH), jnp.int32),
            pltpu.VMEM((1, 2 * EMB), jnp.float32),
            [pltpu.VMEM((2, CH), jnp.float32) for _ in range(DEPTH)],
            pltpu.VMEM((MAX_SLOTS, 16), jnp.int32),
            pltpu.SemaphoreType.DMA,
            [pltpu.SemaphoreType.DMA for _ in range(DEPTH)],
            [pltpu.SemaphoreType.DMA for _ in range(DEPTH)],
        ],
    )
    def k(xT_h, e1_h, p1_h, p2f_h, w_h, meta_h, out_h,
          t1, g2, idx1, idx2, wv, acc, ms, sstage, sg, so):
        wid = lax.axis_index("s") * 2 + lax.axis_index("c")
        pltpu.sync_copy(meta_h.at[wid], ms)
        zeros16 = jnp.zeros((16,), jnp.float32)
        one16 = jnp.ones((16,), jnp.int32)
        iota16 = lax.iota(jnp.int32, 16)

        def start_gather(c, j):
            pltpu.async_copy(p2f_h.at[idx2.at[c]], g2[j], sg[j])

        def wait_gather(c, j):
            pltpu.make_async_copy(p2f_h.at[idx2.at[c]], g2[j], sg[j]).wait()

        def start_wb(tid, c, j):
            pltpu.async_copy(acc[j], out_h.at[tid, :, pl.ds(c * CH, CH)], so[j])

        def wait_wb(tid, c, j):
            pltpu.make_async_copy(
                acc[j], out_h.at[tid, :, pl.ds(c * CH, CH)], so[j]).wait()

        def compute_chunk(c, j, pairwise):
            wch = [wv[0, pl.ds(q * 16, 16)] for q in range(8)]

            def group_body(g, _g):
                rows1 = idx1[c, pl.ds(g * 16, 16)]
                av0 = jnp.zeros((16,), jnp.float32)
                av1 = jnp.zeros((16,), jnp.float32)
                ce = jnp.zeros((16,), jnp.int32)
                if pairwise:
                    rows2 = iota16 + g * 16
                for e in range(EMB):
                    v1 = plsc.load_gather(t1, [rows1, ce])
                    if pairwise:
                        v2 = plsc.load_gather(g2[j], [rows2, ce])
                        prod = jnp.maximum(v1 * v2, 0.0)
                    else:
                        prod = jnp.maximum(v1, 0.0)
                    q, l = divmod(e, 16)
                    av0 = av0 + prod * wch[q][l]
                    av1 = av1 + prod * wch[q + 4][l]
                    ce = ce + one16
                acc[j][0, pl.ds(g * 16, 16)] = av0
                acc[j][1, pl.ds(g * 16, 16)] = av1
                return _g

            lax.fori_loop(0, CGRP, group_body, None)

        def run_pair(tid, col1, col2, tbl):
            pltpu.async_copy(w_h.at[tbl + F_FIELDS], wv, sstage)
            pltpu.async_copy(p1_h.at[tbl], t1, sstage)
            pltpu.sync_copy(xT_h.at[col1], idx1)
            pltpu.sync_copy(xT_h.at[col2], idx2)
            base16 = jnp.full((16,), tbl * VOCAB, jnp.int32)

            def bias_body(i, _):
                r = lax.shift_right_logical(i, 3)
                q = lax.bitwise_and(i, 7)
                sl = pl.ds(q * 16, 16)
                idx2[r, sl] = idx2[r, sl] + base16
                return _

            lax.fori_loop(0, NCHUNK * 8, bias_body, None)
            for j in range(DEPTH - 1):
                start_gather(j, j)
            pltpu.make_async_copy(w_h.at[tbl + F_FIELDS], wv, sstage).wait()
            pltpu.make_async_copy(p1_h.at[tbl], t1, sstage).wait()

            def quad_body(i, _):
                for j in range(DEPTH):
                    c = DEPTH * i + j

                    @pl.when(c + DEPTH - 1 < NCHUNK)
                    def _pref(c=c, j=j):
                        start_gather(c + DEPTH - 1, (j + DEPTH - 1) % DEPTH)

                    wait_gather(c, j)

                    @pl.when(i > 0)
                    def _wwb(c=c, j=j):
                        wait_wb(tid, c - DEPTH, j)

                    compute_chunk(c, j, True)
                    start_wb(tid, c, j)
                return _

            lax.fori_loop(0, NCHUNK // DEPTH, quad_body, None)
            for j in range(DEPTH):
                wait_wb(tid, NCHUNK - DEPTH + j, j)

        def run_field(tid, col1, tbl):
            pltpu.async_copy(w_h.at[tbl], wv, sstage)
            pltpu.async_copy(e1_h.at[tbl], t1, sstage)
            pltpu.sync_copy(xT_h.at[col1], idx1)
            pltpu.make_async_copy(w_h.at[tbl], wv, sstage).wait()
            pltpu.make_async_copy(e1_h.at[tbl], t1, sstage).wait()

            def quad_body(i, _):
                for j in range(DEPTH):
                    c = DEPTH * i + j

                    @pl.when(i > 0)
                    def _wwb(c=c, j=j):
                        wait_wb(tid, c - DEPTH, j)

                    compute_chunk(c, j, False)
                    start_wb(tid, c, j)
                return _

            lax.fori_loop(0, NCHUNK // DEPTH, quad_body, None)
            for j in range(DEPTH):
                wait_wb(tid, NCHUNK - DEPTH + j, j)

        def task_body(slot, _):
            mrow = ms[slot, pl.ds(0, 16)]
            col1 = mrow[0]
            col2 = mrow[1]
            tbl = mrow[2]
            seg = mrow[3]
            tid = slot * NW + wid

            @pl.when(col2 >= 0)
            def _pair():
                run_pair(tid, col1, col2, tbl)

            @pl.when(jnp.logical_and(seg >= 0, col2 < 0))
            def _field():
                run_field(tid, col1, tbl)

            @pl.when(seg < 0)
            def _idle():
                def zg(g, _g):
                    acc[0][0, pl.ds(g * 16, 16)] = zeros16
                    acc[0][1, pl.ds(g * 16, 16)] = zeros16
                    return _g

                lax.fori_loop(0, CGRP, zg, None)

                def zc(ci, _c):
                    pltpu.sync_copy(acc[0], out_h.at[tid, :, pl.ds(ci * CH, CH)])
                    return _c

                lax.fori_loop(0, NCHUNK, zc, None)

            return _

        lax.fori_loop(0, MAX_SLOTS, task_body, None)

    return k(xT3, emb1_tables, pair_emb1, pair_emb2f, Wseg, meta)


def _tc_finish(partials, bias):
    nblk = 32
    nsteps = NTP // nblk

    def body(p_ref, b_ref, o_ref):
        i = pl.program_id(0)

        @pl.when(i == 0)
        def _init():
            o_ref[...] = jnp.zeros_like(o_ref)

        o_ref[...] += jnp.sum(p_ref[...], axis=0)

        @pl.when(i == nsteps - 1)
        def _fin():
            o_ref[...] = jax.nn.sigmoid(o_ref[...] + b_ref[...])

    return pl.pallas_call(
        body,
        grid=(nsteps,),
        in_specs=[
            pl.BlockSpec((nblk, 2, B), lambda i: (i, 0, 0)),
            pl.BlockSpec((2, 1), lambda i: (0, 0)),
        ],
        out_specs=pl.BlockSpec((2, B), lambda i: (0, 0)),
        out_shape=jax.ShapeDtypeStruct((2, B), jnp.float32),
    )(partials, bias)


def kernel(x, emb1_tables, pair_emb1, pair_emb2, W, b):
    xT3 = x.astype(jnp.int32).T.reshape(F_FIELDS, NCHUNK, CH)
    Wseg = W.reshape(2, NSEG, EMB).transpose(1, 0, 2).reshape(NSEG, 1, 2 * EMB)
    pair_emb2f = pair_emb2.reshape(NPAIR * VOCAB, EMB)
    meta = jnp.asarray(_TASK_META)
    partials = _sc_partials(xT3, emb1_tables, pair_emb1, pair_emb2f, Wseg, meta)
    y = _tc_finish(partials, b.reshape(2, 1))
    return (y[0], y[1])


# speedup vs baseline: 1.8965x; 1.2240x over previous
import functools

import jax
import jax.numpy as jnp
import numpy as np
from jax import lax
from jax.experimental import pallas as pl
from jax.experimental.pallas import tpu as pltpu
from jax.experimental.pallas import tpu_sc as plsc

F_FIELDS = 26
EMB = 64
EMBW = EMB // 2
VOCAB = 1000
_pairs = [(i, j) for i in range(F_FIELDS) for j in range(i, F_FIELDS)]
NPAIR = len(_pairs)
NSEG = F_FIELDS + NPAIR
B = 4096
NW = 32
MAX_SLOTS = (NSEG + NW - 1) // NW
NTP = NW * MAX_SLOTS
CH = 512
NCHUNK = B // CH
CGRP = CH // 16
DEPTH = 4


def _build_task_meta():
    meta = np.full((NW, MAX_SLOTS, 16), -1, dtype=np.int32)
    tasks = []
    for f in range(F_FIELDS):
        tasks.append((f, -1, f, f))
    for p, (i, j) in enumerate(_pairs):
        tasks.append((i, j, p, F_FIELDS + p))
    for t, (c1, c2, tbl, seg) in enumerate(tasks):
        meta[t % NW, t // NW, :4] = (c1, c2, tbl, seg)
    return meta


_TASK_META = _build_task_meta()


def _sc_partials(xT3, emb1_tables, pair_emb1, pair_emb2w, Wseg, meta):
    mesh = plsc.VectorSubcoreMesh(core_axis_name="c", subcore_axis_name="s")

    @functools.partial(
        pl.kernel,
        out_type=jax.ShapeDtypeStruct((NTP, 2, B), jnp.float32),
        mesh=mesh,
        compiler_params=pltpu.CompilerParams(
            use_tc_tiling_on_sc=False, needs_layout_passes=False),
        scratch_types=[
            pltpu.VMEM((VOCAB, EMB), jnp.float32),
            pltpu.VMEM((VOCAB, EMBW), jnp.int32),
            pltpu.VMEM((NCHUNK, CH), jnp.int32),
            pltpu.VMEM((NCHUNK, CH), jnp.int32),
            pltpu.VMEM((1, 2 * EMB), jnp.float32),
            [pltpu.VMEM((2, CH), jnp.float32) for _ in range(DEPTH)],
            pltpu.VMEM((MAX_SLOTS, 16), jnp.int32),
            pltpu.SemaphoreType.DMA,
            [pltpu.SemaphoreType.DMA for _ in range(DEPTH)],
        ],
    )
    def k(xT_h, e1_h, p1_h, p2w_h, w_h, meta_h, out_h,
          t1, t2, idx1, idx2, wv, acc, ms, sstage, so):
        wid = lax.axis_index("s") * 2 + lax.axis_index("c")
        pltpu.sync_copy(meta_h.at[wid], ms)
        zeros16 = jnp.zeros((16,), jnp.float32)
        one16 = jnp.ones((16,), jnp.int32)

        def start_wb(tid, c, j):
            pltpu.async_copy(acc[j], out_h.at[tid, :, pl.ds(c * CH, CH)], so[j])

        def wait_wb(tid, c, j):
            pltpu.make_async_copy(
                acc[j], out_h.at[tid, :, pl.ds(c * CH, CH)], so[j]).wait()

        def compute_chunk(c, j, pairwise):
            def group_body(g, _g):
                rows1 = idx1[c, pl.ds(g * 16, 16)]
                av0 = jnp.zeros((16,), jnp.float32)
                av1 = jnp.zeros((16,), jnp.float32)
                if pairwise:
                    rows2 = idx2[c, pl.ds(g * 16, 16)]
                    cw = jnp.zeros((16,), jnp.int32)
                ce = jnp.zeros((16,), jnp.int32)
                for q in range(4):
                    w0q = wv[0, pl.ds(q * 16, 16)]
                    w1q = wv[0, pl.ds(EMB + q * 16, 16)]
                    for l8 in range(8):
                        a0 = plsc.load_gather(t1, [rows1, ce])
                        ce = ce + one16
                        a1 = plsc.load_gather(t1, [rows1, ce])
                        ce = ce + one16
                        if pairwise:
                            bw = plsc.load_gather(t2, [rows2, cw])
                            cw = cw + one16
                            bb = plsc.bitcast(bw, jnp.bfloat16)
                            be, bo = plsc.unpack(
                                bb, format=plsc.PackFormat.INTERLEAVED)
                            pe = jnp.maximum(a0 * be, 0.0)
                            po = jnp.maximum(a1 * bo, 0.0)
                        else:
                            pe = jnp.maximum(a0, 0.0)
                            po = jnp.maximum(a1, 0.0)
                        av0 = av0 + pe * w0q[2 * l8] + po * w0q[2 * l8 + 1]
                        av1 = av1 + pe * w1q[2 * l8] + po * w1q[2 * l8 + 1]
                acc[j][0, pl.ds(g * 16, 16)] = av0
                acc[j][1, pl.ds(g * 16, 16)] = av1
                return _g

            lax.fori_loop(0, CGRP, group_body, None)

        def run_task(tid, col1, col2, tbl, pairwise):
            if pairwise:
                pltpu.async_copy(w_h.at[tbl + F_FIELDS], wv, sstage)
                pltpu.async_copy(p1_h.at[tbl], t1, sstage)
                pltpu.async_copy(p2w_h.at[tbl], t2, sstage)
                pltpu.sync_copy(xT_h.at[col1], idx1)
                pltpu.sync_copy(xT_h.at[col2], idx2)
                pltpu.make_async_copy(w_h.at[tbl + F_FIELDS], wv, sstage).wait()
                pltpu.make_async_copy(p1_h.at[tbl], t1, sstage).wait()
                pltpu.make_async_copy(p2w_h.at[tbl], t2, sstage).wait()
            else:
                pltpu.async_copy(w_h.at[tbl], wv, sstage)
                pltpu.async_copy(e1_h.at[tbl], t1, sstage)
                pltpu.sync_copy(xT_h.at[col1], idx1)
                pltpu.make_async_copy(w_h.at[tbl], wv, sstage).wait()
                pltpu.make_async_copy(e1_h.at[tbl], t1, sstage).wait()

            def quad_body(i, _):
                for j in range(DEPTH):
                    c = DEPTH * i + j

                    @pl.when(i > 0)
                    def _wwb(c=c, j=j):
                        wait_wb(tid, c - DEPTH, j)

                    compute_chunk(c, j, pairwise)
                    start_wb(tid, c, j)
                return _

            lax.fori_loop(0, NCHUNK // DEPTH, quad_body, None)
            for j in range(DEPTH):
                wait_wb(tid, NCHUNK - DEPTH + j, j)

        def task_body(slot, _):
            mrow = ms[slot, pl.ds(0, 16)]
            col1 = mrow[0]
            col2 = mrow[1]
            tbl = mrow[2]
            seg = mrow[3]
            tid = slot * NW + wid

            @pl.when(col2 >= 0)
            def _pair():
                run_task(tid, col1, col2, tbl, True)

            @pl.when(jnp.logical_and(seg >= 0, col2 < 0))
            def _field():
                run_task(tid, col1, col2, tbl, False)

            @pl.when(seg < 0)
            def _idle():
                def zg(g, _g):
                    acc[0][0, pl.ds(g * 16, 16)] = zeros16
                    acc[0][1, pl.ds(g * 16, 16)] = zeros16
                    return _g

                lax.fori_loop(0, CGRP, zg, None)

                def zc(ci, _c):
                    pltpu.sync_copy(acc[0], out_h.at[tid, :, pl.ds(ci * CH, CH)])
                    return _c

                lax.fori_loop(0, NCHUNK, zc, None)

            return _

        lax.fori_loop(0, MAX_SLOTS, task_body, None)

    return k(xT3, emb1_tables, pair_emb1, pair_emb2w, Wseg, meta)


def _tc_finish(partials, bias):
    nblk = 32
    nsteps = NTP // nblk

    def body(p_ref, b_ref, o_ref):
        i = pl.program_id(0)

        @pl.when(i == 0)
        def _init():
            o_ref[...] = jnp.zeros_like(o_ref)

        o_ref[...] += jnp.sum(p_ref[...], axis=0)

        @pl.when(i == nsteps - 1)
        def _fin():
            o_ref[...] = jax.nn.sigmoid(o_ref[...] + b_ref[...])

    return pl.pallas_call(
        body,
        grid=(nsteps,),
        in_specs=[
            pl.BlockSpec((nblk, 2, B), lambda i: (i, 0, 0)),
            pl.BlockSpec((2, 1), lambda i: (0, 0)),
        ],
        out_specs=pl.BlockSpec((2, B), lambda i: (0, 0)),
        out_shape=jax.ShapeDtypeStruct((2, B), jnp.float32),
    )(partials, bias)


def kernel(x, emb1_tables, pair_emb1, pair_emb2, W, b):
    xT3 = x.astype(jnp.int32).T.reshape(F_FIELDS, NCHUNK, CH)
    Wseg = W.reshape(2, NSEG, EMB).transpose(1, 0, 2).reshape(NSEG, 1, 2 * EMB)
    pair_emb2w = jax.lax.bitcast_convert_type(
        pair_emb2.astype(jnp.bfloat16).reshape(NPAIR, VOCAB, EMBW, 2),
        jnp.int32)
    meta = jnp.asarray(_TASK_META)
    partials = _sc_partials(xT3, emb1_tables, pair_emb1, pair_emb2w, Wseg, meta)
    y = _tc_finish(partials, b.reshape(2, 1))
    return (y[0], y[1])
